# baseline (speedup 1.0000x reference)
import functools

import jax
import jax.numpy as jnp
from jax import lax
from jax.experimental import pallas as pl
from jax.experimental.pallas import tpu as pltpu
from jax.experimental.pallas import tpu_sc as plsc

_SCALE = 30.0
_EPS = 1e-5

_NC = 2
_NS = 16
_NW = _NC * _NS
_CH = 128

_KT = 10


def _gelu(x):
    return 0.5 * x * (1.0 + lax.erf(x * 0.7071067811865476))


def _ln(x, g, b):
    m = jnp.mean(x, axis=-1, keepdims=True)
    v = jnp.mean((x - m) ** 2, axis=-1, keepdims=True)
    return (x - m) * lax.rsqrt(v + _EPS) * g + b


def _sc_gather(table, idx3, n_total, h, ch, pad):
    _CH = ch
    n_ch = idx3.shape[1] - pad
    per_w = n_ch * _CH
    mesh = plsc.VectorSubcoreMesh(core_axis_name="c", subcore_axis_name="s")

    @functools.partial(
        pl.kernel,
        mesh=mesh,
        out_type=jax.ShapeDtypeStruct((n_total, h), jnp.float32),
        compiler_params=pltpu.CompilerParams(use_tc_tiling_on_sc=True),
        scratch_types=(
            [pltpu.VMEM((n_ch + pad, _CH), jnp.int32)]
            + [pltpu.VMEM((_CH, h), jnp.float32)] * 6
            + [pltpu.SemaphoreType.DMA] * 12
        ),
    )
    def gather_k(table_hbm, idx_hbm, out_hbm, idx_v, *bufs_sems):
        wid = lax.axis_index("s") * _NC + lax.axis_index("c")
        base = wid * per_w
        pltpu.sync_copy(idx_hbm.at[wid], idx_v)

        bufs = bufs_sems[:6]
        gs = bufs_sems[6:12]
        ps = bufs_sems[12:18]

        def gather(j, s, sem):
            return pltpu.make_async_copy(table_hbm.at[idx_v.at[j]], bufs[s], sem)

        def push(j, s, sem):
            return pltpu.make_async_copy(
                bufs[s], out_hbm.at[pl.ds(base + j * _CH, _CH)], sem
            )

        gather(0, 0, gs[0]).start()
        gather(1, 1, gs[1]).start()
        m = (n_ch - 6) // 6

        def body(i, carry):
            j0 = 6 * i
            for d in range(6):
                j = j0 + d
                s2 = (d + 2) % 6

                def _drain(j=j, s2=s2):
                    push(j - 4, s2, ps[s2]).wait()

                if d < 4:
                    pl.when(j0 >= 4)(_drain)
                else:
                    _drain()
                gather(j + 2, s2, gs[s2]).start()
                gather(j, d, gs[d]).wait()
                push(j, d, ps[d]).start()
            return carry

        lax.fori_loop(0, m, body, 0)
        jt = 6 * m
        for d in range(6):
            j = jt + d
            s2 = (d + 2) % 6
            if j - 4 >= 0:
                push(j - 4, s2, ps[s2]).wait()
            if j + 2 < n_ch:
                gather(j + 2, s2, gs[s2]).start()
            gather(j, d, gs[d]).wait()
            push(j, d, ps[d]).start()
        for d in range(4):
            j = n_ch - 4 + d
            push(j, j % 6, ps[j % 6]).wait()

    return gather_k(table, idx3)


def _kernel_a(hv_ref, w1a_ref, b1_ref, w1c_ref, a1_ref, p1_ref):
    x = hv_ref[...]
    a1_ref[...] = (
        jnp.dot(x, w1a_ref[...], preferred_element_type=jnp.float32) + b1_ref[...]
    )
    p1_ref[...] = jnp.dot(x, w1c_ref[...], preferred_element_type=jnp.float32)


def _kernel_b(
    hv_ref, he_ref, g1_ref, a1_ref,
    w1b_ref, w2_ref, b2_ref, w3_ref, b3_ref,
    win_ref, bin_ref, wout_ref, bout_ref,
    n1g_ref, n1b_ref, n2g_ref, n2b_ref,
    w11a_ref, b11_ref, w11c_ref,
    hv_out_ref, p2_ref, a2_ref,
    acc_ref,
    *, kt, n_kt, ll,
):
    h = hv_ref.shape[-1]
    step = pl.program_id(1)
    he = he_ref[0].reshape(kt * ll, h)
    z = jnp.dot(he, w1b_ref[...], preferred_element_type=jnp.float32) + g1_ref[
        0
    ].reshape(kt * ll, h)
    z = z.reshape(kt, ll, h) + a1_ref[0][None]
    m = _gelu(z).reshape(kt * ll, h)
    m = _gelu(jnp.dot(m, w2_ref[...], preferred_element_type=jnp.float32) + b2_ref[...])
    m = jnp.dot(m, w3_ref[...], preferred_element_type=jnp.float32) + b3_ref[...]
    part = jnp.sum(m.reshape(kt, ll, h), axis=0)

    @pl.when(step == 0)
    def _():
        acc_ref[...] = part

    @pl.when(step > 0)
    def _():
        acc_ref[...] += part

    @pl.when(step == n_kt - 1)
    def _():
        dh = acc_ref[...] * (1.0 / _SCALE)
        h1 = _ln(hv_ref[0] + dh, n1g_ref[...], n1b_ref[...])
        f = _gelu(
            jnp.dot(h1, win_ref[...], preferred_element_type=jnp.float32)
            + bin_ref[...]
        )
        d2 = (
            jnp.dot(f, wout_ref[...], preferred_element_type=jnp.float32)
            + bout_ref[...]
        )
        h2 = _ln(h1 + d2, n2g_ref[...], n2b_ref[...])
        hv_out_ref[0] = h2
        p2_ref[0] = jnp.dot(h2, w11c_ref[...], preferred_element_type=jnp.float32)
        a2_ref[0] = (
            jnp.dot(h2, w11a_ref[...], preferred_element_type=jnp.float32)
            + b11_ref[...]
        )


def _kernel_c(
    he_ref, g2_ref, a2_ref,
    w11b_ref, w12_ref, b12_ref, w13_ref, b13_ref,
    n3g_ref, n3b_ref,
    he_out_ref,
    *, kt, ll,
):
    h = a2_ref.shape[-1]
    he = he_ref[0].reshape(kt * ll, h)
    z = jnp.dot(he, w11b_ref[...], preferred_element_type=jnp.float32) + g2_ref[
        0
    ].reshape(kt * ll, h)
    z = z.reshape(kt, ll, h) + a2_ref[0][None]
    m = _gelu(z).reshape(kt * ll, h)
    m = _gelu(
        jnp.dot(m, w12_ref[...], preferred_element_type=jnp.float32) + b12_ref[...]
    )
    m = jnp.dot(m, w13_ref[...], preferred_element_type=jnp.float32) + b13_ref[...]
    he_out_ref[0] = (_ln(he + m, n3g_ref[...], n3b_ref[...])).reshape(kt, ll, h)


def _row_spec(h):
    return pl.BlockSpec((1, h), lambda *_: (0, 0))


def kernel(h_V, h_E, E_idx, params):
    b, l, h = h_V.shape
    k = E_idx.shape[-1]
    n = b * l
    e_total = n * k
    n_kt = k // _KT
    p = params

    def t(name):
        return p[name].T

    def row(name):
        return p[name].reshape(1, -1)

    w1 = p["W1_w"]
    w1a_t, w1b_t, w1c_t = w1[:, :h].T, w1[:, h : 2 * h].T, w1[:, 2 * h :].T
    w11 = p["W11_w"]
    w11a_t, w11b_t, w11c_t = w11[:, :h].T, w11[:, h : 2 * h].T, w11[:, 2 * h :].T

    hv_flat = h_V.reshape(n, h)

    a1, p1 = pl.pallas_call(
        _kernel_a,
        out_shape=(
            jax.ShapeDtypeStruct((n, h), jnp.float32),
            jax.ShapeDtypeStruct((n, h), jnp.float32),
        ),
    )(hv_flat, w1a_t, row("W1_b"), w1c_t)

    he_t = h_E.transpose(0, 2, 1, 3)
    a1r = a1.reshape(b, l, h)

    glob_idx = E_idx.astype(jnp.int32).transpose(0, 2, 1) + (
        jnp.arange(b, dtype=jnp.int32) * l
    )[:, None, None]
    ch2 = _CH
    n_ch2 = e_total // (_NW * ch2)
    pad2 = (-n_ch2) % 8
    idx_full = jnp.pad(
        glob_idx.reshape(_NW, n_ch2, ch2), ((0, 0), (0, pad2), (0, 0))
    )

    w_spec = lambda arr: pl.BlockSpec(arr.shape, lambda *_: (0,) * arr.ndim)

    hb = 2
    ch1 = 80
    n_ch1 = hb * l * k // (_NW * ch1)
    idx_half = glob_idx.reshape(b // hb, _NW, n_ch1, ch1)

    edge_spec = pl.BlockSpec((1, _KT, l, h), lambda i, j: (i, j, 0, 0))
    node_spec = pl.BlockSpec((1, l, h), lambda i, j: (i, 0, 0))

    hv_parts, p2_parts, a2_parts = [], [], []
    for half in range(b // hb):
        s = slice(half * hb, (half + 1) * hb)
        g1_h = _sc_gather(p1, idx_half[half], hb * l * k, h, ch1, 0)
        hv_h, p2_h, a2_h = pl.pallas_call(
            functools.partial(_kernel_b, kt=_KT, n_kt=n_kt, ll=l),
            grid=(hb, n_kt),
            in_specs=[
                node_spec, edge_spec, edge_spec, node_spec,
                w_spec(w1b_t), w_spec(t("W2_w")), _row_spec(h),
                w_spec(t("W3_w")), _row_spec(h),
                w_spec(t("Win_w")), _row_spec(4 * h),
                w_spec(t("Wout_w")), _row_spec(h),
                _row_spec(h), _row_spec(h), _row_spec(h), _row_spec(h),
                w_spec(w11a_t), _row_spec(h),
                w_spec(w11c_t),
            ],
            out_specs=(node_spec, node_spec, node_spec),
            out_shape=(
                jax.ShapeDtypeStruct((hb, l, h), jnp.float32),
                jax.ShapeDtypeStruct((hb, l, h), jnp.float32),
                jax.ShapeDtypeStruct((hb, l, h), jnp.float32),
            ),
            scratch_shapes=[pltpu.VMEM((l, h), jnp.float32)],
            compiler_params=pltpu.CompilerParams(
                dimension_semantics=("arbitrary", "arbitrary"),
            ),
        )(
            h_V[s], he_t[s], g1_h.reshape(hb, k, l, h), a1r[s],
            w1b_t, t("W2_w"), row("W2_b"), t("W3_w"), row("W3_b"),
            t("Win_w"), row("Win_b"), t("Wout_w"), row("Wout_b"),
            row("norm1_g"), row("norm1_b"), row("norm2_g"), row("norm2_b"),
            w11a_t, row("W11_b"), w11c_t,
        )
        hv_parts.append(hv_h)
        p2_parts.append(p2_h)
        a2_parts.append(a2_h)

    hv_out = jnp.concatenate(hv_parts)
    p2 = jnp.concatenate(p2_parts)
    a2 = jnp.concatenate(a2_parts)

    g2 = _sc_gather(p2.reshape(n, h), idx_full, e_total, h, ch2, pad2)
    g2r = g2.reshape(b, k, l, h)

    he_out = pl.pallas_call(
        functools.partial(_kernel_c, kt=_KT, ll=l),
        grid=(b, n_kt),
        in_specs=[
            edge_spec, edge_spec, node_spec,
            w_spec(w11b_t), w_spec(t("W12_w")), _row_spec(h),
            w_spec(t("W13_w")), _row_spec(h),
            _row_spec(h), _row_spec(h),
        ],
        out_specs=edge_spec,
        out_shape=jax.ShapeDtypeStruct((b, k, l, h), jnp.float32),
        compiler_params=pltpu.CompilerParams(
            dimension_semantics=("arbitrary", "arbitrary"),
        ),
    )(
        he_t, g2r, a2,
        w11b_t, t("W12_w"), row("W12_b"), t("W13_w"), row("W13_b"),
        row("norm3_g"), row("norm3_b"),
    )

    return hv_out, he_out.transpose(0, 2, 1, 3)


# speedup vs baseline: 1.1978x; 1.1978x over previous
import functools

import jax
import jax.numpy as jnp
from jax import lax
from jax.experimental import pallas as pl
from jax.experimental.pallas import tpu as pltpu
from jax.experimental.pallas import tpu_sc as plsc

_SCALE = 30.0
_EPS = 1e-5

_NC = 2
_NS = 16
_NW = _NC * _NS
_CH = 128

_KT = 15


def _gelu(x):
    return 0.5 * x * (1.0 + lax.erf(x * 0.7071067811865476))


def _ln(x, g, b):
    m = jnp.mean(x, axis=-1, keepdims=True)
    v = jnp.mean((x - m) ** 2, axis=-1, keepdims=True)
    return (x - m) * lax.rsqrt(v + _EPS) * g + b


def _sc_gather(table, idx3, n_total, h, ch, pad):
    _CH = ch
    n_ch = idx3.shape[1] - pad
    per_w = n_ch * _CH
    mesh = plsc.VectorSubcoreMesh(core_axis_name="c", subcore_axis_name="s")

    @functools.partial(
        pl.kernel,
        mesh=mesh,
        out_type=jax.ShapeDtypeStruct((n_total, h), jnp.float32),
        compiler_params=pltpu.CompilerParams(use_tc_tiling_on_sc=True),
        scratch_types=[
            pltpu.VMEM((n_ch + pad, _CH), jnp.int32),
            pltpu.VMEM((_CH, h), jnp.float32),
            pltpu.VMEM((_CH, h), jnp.float32),
            pltpu.VMEM((_CH, h), jnp.float32),
            pltpu.VMEM((_CH, h), jnp.float32),
            pltpu.SemaphoreType.DMA,
            pltpu.SemaphoreType.DMA,
            pltpu.SemaphoreType.DMA,
            pltpu.SemaphoreType.DMA,
            pltpu.SemaphoreType.DMA,
            pltpu.SemaphoreType.DMA,
            pltpu.SemaphoreType.DMA,
            pltpu.SemaphoreType.DMA,
        ],
    )
    def gather_k(
        table_hbm, idx_hbm, out_hbm, idx_v,
        buf0, buf1, buf2, buf3,
        gs0, gs1, gs2, gs3, ps0, ps1, ps2, ps3,
    ):
        wid = lax.axis_index("s") * _NC + lax.axis_index("c")
        base = wid * per_w
        pltpu.sync_copy(idx_hbm.at[wid], idx_v)

        bufs = (buf0, buf1, buf2, buf3)
        gs = (gs0, gs1, gs2, gs3)
        ps = (ps0, ps1, ps2, ps3)

        def gather(j, s, sem):
            return pltpu.make_async_copy(table_hbm.at[idx_v.at[j]], bufs[s], sem)

        def push(j, s, sem):
            return pltpu.make_async_copy(
                bufs[s], out_hbm.at[pl.ds(base + j * _CH, _CH)], sem
            )

        gather(0, 0, gs[0]).start()
        gather(1, 1, gs[1]).start()
        m = (n_ch - 2) // 4

        def body(i, carry):
            j0 = 4 * i
            for d in range(4):
                j = j0 + d
                s2 = (d + 2) % 4

                def _advance(j=j, s2=s2):
                    push(j - 2, s2, ps[s2]).wait()
                    gather(j + 2, s2, gs[s2]).start()

                if d < 2:
                    pl.when(j0 >= 2)(_advance)

                    @pl.when(j0 < 2)
                    def _(j=j, s2=s2):
                        gather(j + 2, s2, gs[s2]).start()

                else:
                    _advance()
                gather(j, d, gs[d]).wait()
                push(j, d, ps[d]).start()
            return carry

        lax.fori_loop(0, m, body, 0)
        jt = 4 * m
        for d in range(2):
            gather(jt + d, d, gs[d]).wait()
            push(jt + d, d, ps[d]).start()
        for j, s in ((jt - 2, 2), (jt - 1, 3), (jt, 0), (jt + 1, 1)):
            push(j, s, ps[s]).wait()

    return gather_k(table, idx3)


def _kernel_a(hv_ref, w1a_ref, b1_ref, w1c_ref, a1_ref, p1_ref):
    x = hv_ref[...]
    a1_ref[...] = (
        jnp.dot(x, w1a_ref[...], preferred_element_type=jnp.float32) + b1_ref[...]
    )
    p1_ref[...] = jnp.dot(x, w1c_ref[...], preferred_element_type=jnp.float32)


def _kernel_b(
    hv_ref, he_ref, g1_ref, a1_ref,
    w1b_ref, w2_ref, b2_ref, w3_ref, b3_ref,
    win_ref, bin_ref, wout_ref, bout_ref,
    n1g_ref, n1b_ref, n2g_ref, n2b_ref,
    w11a_ref, b11_ref, w11c_ref,
    hv_out_ref, p2_ref, a2_ref,
    acc_ref,
    *, kt, n_kt, ll,
):
    h = hv_ref.shape[-1]
    step = pl.program_id(1)
    he = he_ref[0].reshape(kt * ll, h)
    z = jnp.dot(he, w1b_ref[...], preferred_element_type=jnp.float32) + g1_ref[
        0
    ].reshape(kt * ll, h)
    z = z.reshape(kt, ll, h) + a1_ref[0][None]
    m = _gelu(z).reshape(kt * ll, h)
    m = _gelu(jnp.dot(m, w2_ref[...], preferred_element_type=jnp.float32) + b2_ref[...])
    m = jnp.dot(m, w3_ref[...], preferred_element_type=jnp.float32) + b3_ref[...]
    part = jnp.sum(m.reshape(kt, ll, h), axis=0)

    @pl.when(step == 0)
    def _():
        acc_ref[...] = part

    @pl.when(step > 0)
    def _():
        acc_ref[...] += part

    @pl.when(step == n_kt - 1)
    def _():
        dh = acc_ref[...] * (1.0 / _SCALE)
        h1 = _ln(hv_ref[0] + dh, n1g_ref[...], n1b_ref[...])
        f = _gelu(
            jnp.dot(h1, win_ref[...], preferred_element_type=jnp.float32)
            + bin_ref[...]
        )
        d2 = (
            jnp.dot(f, wout_ref[...], preferred_element_type=jnp.float32)
            + bout_ref[...]
        )
        h2 = _ln(h1 + d2, n2g_ref[...], n2b_ref[...])
        hv_out_ref[0] = h2
        p2_ref[0] = jnp.dot(h2, w11c_ref[...], preferred_element_type=jnp.float32)
        a2_ref[0] = (
            jnp.dot(h2, w11a_ref[...], preferred_element_type=jnp.float32)
            + b11_ref[...]
        )


def _kernel_c(
    he_ref, g2_ref, a2_ref,
    w11b_ref, w12_ref, b12_ref, w13_ref, b13_ref,
    n3g_ref, n3b_ref,
    he_out_ref,
    *, kt, ll,
):
    h = a2_ref.shape[-1]
    he = he_ref[0].reshape(kt * ll, h)
    z = jnp.dot(he, w11b_ref[...], preferred_element_type=jnp.float32) + g2_ref[
        0
    ].reshape(kt * ll, h)
    z = z.reshape(kt, ll, h) + a2_ref[0][None]
    m = _gelu(z).reshape(kt * ll, h)
    m = _gelu(
        jnp.dot(m, w12_ref[...], preferred_element_type=jnp.float32) + b12_ref[...]
    )
    m = jnp.dot(m, w13_ref[...], preferred_element_type=jnp.float32) + b13_ref[...]
    he_out_ref[0] = (_ln(he + m, n3g_ref[...], n3b_ref[...])).reshape(kt, ll, h)


def _row_spec(h):
    return pl.BlockSpec((1, h), lambda *_: (0, 0))


def kernel(h_V, h_E, E_idx, params):
    b, l, h = h_V.shape
    k = E_idx.shape[-1]
    n = b * l
    e_total = n * k
    n_kt = k // _KT
    p = params

    def t(name):
        return p[name].T

    def row(name):
        return p[name].reshape(1, -1)

    w1 = p["W1_w"]
    w1a_t, w1b_t, w1c_t = w1[:, :h].T, w1[:, h : 2 * h].T, w1[:, 2 * h :].T
    w11 = p["W11_w"]
    w11a_t, w11b_t, w11c_t = w11[:, :h].T, w11[:, h : 2 * h].T, w11[:, 2 * h :].T

    hv_flat = h_V.reshape(n, h)

    a1, p1 = pl.pallas_call(
        _kernel_a,
        out_shape=(
            jax.ShapeDtypeStruct((n, h), jnp.float32),
            jax.ShapeDtypeStruct((n, h), jnp.float32),
        ),
    )(hv_flat, w1a_t, row("W1_b"), w1c_t)

    he_t = h_E.transpose(0, 2, 1, 3)
    a1r = a1.reshape(b, l, h)

    glob_idx = E_idx.astype(jnp.int32).transpose(0, 2, 1) + (
        jnp.arange(b, dtype=jnp.int32) * l
    )[:, None, None]
    ch2 = _CH
    n_ch2 = e_total // (_NW * ch2)
    pad2 = (-n_ch2) % 8
    idx_full = jnp.pad(
        glob_idx.reshape(_NW, n_ch2, ch2), ((0, 0), (0, pad2), (0, 0))
    )

    w_spec = lambda arr: pl.BlockSpec(arr.shape, lambda *_: (0,) * arr.ndim)

    g1 = _sc_gather(p1, idx_full, e_total, h, ch2, pad2)
    g1r = g1.reshape(b, k, l, h)

    grid = (b, n_kt)
    edge_spec = pl.BlockSpec((1, _KT, l, h), lambda i, j: (i, j, 0, 0))
    node_spec = pl.BlockSpec((1, l, h), lambda i, j: (i, 0, 0))

    hv_out, p2, a2 = pl.pallas_call(
        functools.partial(_kernel_b, kt=_KT, n_kt=n_kt, ll=l),
        grid=grid,
        in_specs=[
            node_spec, edge_spec, edge_spec, node_spec,
            w_spec(w1b_t), w_spec(t("W2_w")), _row_spec(h),
            w_spec(t("W3_w")), _row_spec(h),
            w_spec(t("Win_w")), _row_spec(4 * h),
            w_spec(t("Wout_w")), _row_spec(h),
            _row_spec(h), _row_spec(h), _row_spec(h), _row_spec(h),
            w_spec(w11a_t), _row_spec(h),
            w_spec(w11c_t),
        ],
        out_specs=(node_spec, node_spec, node_spec),
        out_shape=(
            jax.ShapeDtypeStruct((b, l, h), jnp.float32),
            jax.ShapeDtypeStruct((b, l, h), jnp.float32),
            jax.ShapeDtypeStruct((b, l, h), jnp.float32),
        ),
        scratch_shapes=[pltpu.VMEM((l, h), jnp.float32)],
        compiler_params=pltpu.CompilerParams(
            dimension_semantics=("arbitrary", "arbitrary"),
        ),
    )(
        h_V, he_t, g1r, a1r,
        w1b_t, t("W2_w"), row("W2_b"), t("W3_w"), row("W3_b"),
        t("Win_w"), row("Win_b"), t("Wout_w"), row("Wout_b"),
        row("norm1_g"), row("norm1_b"), row("norm2_g"), row("norm2_b"),
        w11a_t, row("W11_b"), w11c_t,
    )

    g2 = _sc_gather(p2.reshape(n, h), idx_full, e_total, h, ch2, pad2)
    g2r = g2.reshape(b, k, l, h)

    he_out = pl.pallas_call(
        functools.partial(_kernel_c, kt=_KT, ll=l),
        grid=(b, n_kt),
        in_specs=[
            edge_spec, edge_spec, node_spec,
            w_spec(w11b_t), w_spec(t("W12_w")), _row_spec(h),
            w_spec(t("W13_w")), _row_spec(h),
            _row_spec(h), _row_spec(h),
        ],
        out_specs=edge_spec,
        out_shape=jax.ShapeDtypeStruct((b, k, l, h), jnp.float32),
        compiler_params=pltpu.CompilerParams(
            dimension_semantics=("arbitrary", "arbitrary"),
        ),
    )(
        he_t, g2r, a2,
        w11b_t, t("W12_w"), row("W12_b"), t("W13_w"), row("W13_b"),
        row("norm3_g"), row("norm3_b"),
    )

    return hv_out, he_out.transpose(0, 2, 1, 3)


# speedup vs baseline: 1.2059x; 1.0067x over previous
import functools

import jax
import jax.numpy as jnp
from jax import lax
from jax.experimental import pallas as pl
from jax.experimental.pallas import tpu as pltpu
from jax.experimental.pallas import tpu_sc as plsc

_SCALE = 30.0
_EPS = 1e-5

_NC = 2
_NS = 16
_NW = _NC * _NS
_CH = 128

_KT = 10


def _gelu(x):
    return 0.5 * x * (1.0 + lax.erf(x * 0.7071067811865476))


def _ln(x, g, b):
    m = jnp.mean(x, axis=-1, keepdims=True)
    v = jnp.mean((x - m) ** 2, axis=-1, keepdims=True)
    return (x - m) * lax.rsqrt(v + _EPS) * g + b


def _sc_gather(table, idx3, n_total, h, ch, pad):
    _CH = ch
    n_ch = idx3.shape[1] - pad
    per_w = n_ch * _CH
    mesh = plsc.VectorSubcoreMesh(core_axis_name="c", subcore_axis_name="s")

    @functools.partial(
        pl.kernel,
        mesh=mesh,
        out_type=jax.ShapeDtypeStruct((n_total, h), jnp.float32),
        compiler_params=pltpu.CompilerParams(use_tc_tiling_on_sc=True),
        scratch_types=[
            pltpu.VMEM((n_ch + pad, _CH), jnp.int32),
            pltpu.VMEM((_CH, h), jnp.float32),
            pltpu.VMEM((_CH, h), jnp.float32),
            pltpu.VMEM((_CH, h), jnp.float32),
            pltpu.VMEM((_CH, h), jnp.float32),
            pltpu.SemaphoreType.DMA,
            pltpu.SemaphoreType.DMA,
            pltpu.SemaphoreType.DMA,
            pltpu.SemaphoreType.DMA,
            pltpu.SemaphoreType.DMA,
            pltpu.SemaphoreType.DMA,
            pltpu.SemaphoreType.DMA,
            pltpu.SemaphoreType.DMA,
        ],
    )
    def gather_k(
        table_hbm, idx_hbm, out_hbm, idx_v,
        buf0, buf1, buf2, buf3,
        gs0, gs1, gs2, gs3, ps0, ps1, ps2, ps3,
    ):
        wid = lax.axis_index("s") * _NC + lax.axis_index("c")
        base = wid * per_w
        pltpu.sync_copy(idx_hbm.at[wid], idx_v)

        bufs = (buf0, buf1, buf2, buf3)
        gs = (gs0, gs1, gs2, gs3)
        ps = (ps0, ps1, ps2, ps3)

        def gather(j, s, sem):
            return pltpu.make_async_copy(table_hbm.at[idx_v.at[j]], bufs[s], sem)

        def push(j, s, sem):
            return pltpu.make_async_copy(
                bufs[s], out_hbm.at[pl.ds(base + j * _CH, _CH)], sem
            )

        gather(0, 0, gs[0]).start()
        gather(1, 1, gs[1]).start()
        m = (n_ch - 2) // 4

        def body(i, carry):
            j0 = 4 * i
            for d in range(4):
                j = j0 + d
                s2 = (d + 2) % 4

                def _advance(j=j, s2=s2):
                    push(j - 2, s2, ps[s2]).wait()
                    gather(j + 2, s2, gs[s2]).start()

                if d < 2:
                    pl.when(j0 >= 2)(_advance)

                    @pl.when(j0 < 2)
                    def _(j=j, s2=s2):
                        gather(j + 2, s2, gs[s2]).start()

                else:
                    _advance()
                gather(j, d, gs[d]).wait()
                push(j, d, ps[d]).start()
            return carry

        lax.fori_loop(0, m, body, 0)
        jt = 4 * m
        for d in range(2):
            gather(jt + d, d, gs[d]).wait()
            push(jt + d, d, ps[d]).start()
        for j, s in ((jt - 2, 2), (jt - 1, 3), (jt, 0), (jt + 1, 1)):
            push(j, s, ps[s]).wait()

    return gather_k(table, idx3)


def _kernel_a(hv_ref, w1a_ref, b1_ref, w1c_ref, a1_ref, p1_ref):
    x = hv_ref[...]
    a1_ref[...] = (
        jnp.dot(x, w1a_ref[...], preferred_element_type=jnp.float32) + b1_ref[...]
    )
    p1_ref[...] = jnp.dot(x, w1c_ref[...], preferred_element_type=jnp.float32)


def _kernel_b(
    hv_ref, he_ref, g1_ref, a1_ref,
    w1b_ref, w2_ref, b2_ref, w3_ref, b3_ref,
    win_ref, bin_ref, wout_ref, bout_ref,
    n1g_ref, n1b_ref, n2g_ref, n2b_ref,
    w11a_ref, b11_ref, w11c_ref,
    hv_out_ref, p2_ref, a2_ref,
    acc_ref,
    *, kt, n_kt, ll,
):
    h = hv_ref.shape[-1]
    step = pl.program_id(1)
    he = he_ref[0].reshape(kt * ll, h)
    z = jnp.dot(he, w1b_ref[...], preferred_element_type=jnp.float32) + g1_ref[
        0
    ].reshape(kt * ll, h)
    z = z.reshape(kt, ll, h) + a1_ref[0][None]
    m = _gelu(z).reshape(kt * ll, h)
    m = _gelu(jnp.dot(m, w2_ref[...], preferred_element_type=jnp.float32) + b2_ref[...])
    m = jnp.dot(m, w3_ref[...], preferred_element_type=jnp.float32) + b3_ref[...]
    part = jnp.sum(m.reshape(kt, ll, h), axis=0)

    @pl.when(step == 0)
    def _():
        acc_ref[...] = part

    @pl.when(step > 0)
    def _():
        acc_ref[...] += part

    @pl.when(step == n_kt - 1)
    def _():
        dh = acc_ref[...] * (1.0 / _SCALE)
        h1 = _ln(hv_ref[0] + dh, n1g_ref[...], n1b_ref[...])
        f = _gelu(
            jnp.dot(h1, win_ref[...], preferred_element_type=jnp.float32)
            + bin_ref[...]
        )
        d2 = (
            jnp.dot(f, wout_ref[...], preferred_element_type=jnp.float32)
            + bout_ref[...]
        )
        h2 = _ln(h1 + d2, n2g_ref[...], n2b_ref[...])
        hv_out_ref[0] = h2
        p2_ref[0] = jnp.dot(h2, w11c_ref[...], preferred_element_type=jnp.float32)
        a2_ref[0] = (
            jnp.dot(h2, w11a_ref[...], preferred_element_type=jnp.float32)
            + b11_ref[...]
        )


def _kernel_c(
    he_ref, g2_ref, a2_ref,
    w11b_ref, w12_ref, b12_ref, w13_ref, b13_ref,
    n3g_ref, n3b_ref,
    he_out_ref,
    *, kt, ll,
):
    h = a2_ref.shape[-1]
    he = he_ref[0].reshape(kt * ll, h)
    z = jnp.dot(he, w11b_ref[...], preferred_element_type=jnp.float32) + g2_ref[
        0
    ].reshape(kt * ll, h)
    z = z.reshape(kt, ll, h) + a2_ref[0][None]
    m = _gelu(z).reshape(kt * ll, h)
    m = _gelu(
        jnp.dot(m, w12_ref[...], preferred_element_type=jnp.float32) + b12_ref[...]
    )
    m = jnp.dot(m, w13_ref[...], preferred_element_type=jnp.float32) + b13_ref[...]
    he_out_ref[0] = (_ln(he + m, n3g_ref[...], n3b_ref[...])).reshape(kt, ll, h)


def _row_spec(h):
    return pl.BlockSpec((1, h), lambda *_: (0, 0))


def kernel(h_V, h_E, E_idx, params):
    b, l, h = h_V.shape
    k = E_idx.shape[-1]
    n = b * l
    e_total = n * k
    n_kt = k // _KT
    p = params

    def t(name):
        return p[name].T

    def row(name):
        return p[name].reshape(1, -1)

    w1 = p["W1_w"]
    w1a_t, w1b_t, w1c_t = w1[:, :h].T, w1[:, h : 2 * h].T, w1[:, 2 * h :].T
    w11 = p["W11_w"]
    w11a_t, w11b_t, w11c_t = w11[:, :h].T, w11[:, h : 2 * h].T, w11[:, 2 * h :].T

    hv_flat = h_V.reshape(n, h)

    a1, p1 = pl.pallas_call(
        _kernel_a,
        out_shape=(
            jax.ShapeDtypeStruct((n, h), jnp.float32),
            jax.ShapeDtypeStruct((n, h), jnp.float32),
        ),
    )(hv_flat, w1a_t, row("W1_b"), w1c_t)

    he_t = h_E.transpose(0, 2, 1, 3)
    a1r = a1.reshape(b, l, h)

    glob_idx = E_idx.astype(jnp.int32).transpose(0, 2, 1) + (
        jnp.arange(b, dtype=jnp.int32) * l
    )[:, None, None]
    ch2 = _CH
    n_ch2 = e_total // (_NW * ch2)
    pad2 = (-n_ch2) % 8
    idx_full = jnp.pad(
        glob_idx.reshape(_NW, n_ch2, ch2), ((0, 0), (0, pad2), (0, 0))
    )

    w_spec = lambda arr: pl.BlockSpec(arr.shape, lambda *_: (0,) * arr.ndim)

    g1 = _sc_gather(p1, idx_full, e_total, h, ch2, pad2)
    g1r = g1.reshape(b, k, l, h)

    grid = (b, n_kt)
    edge_spec = pl.BlockSpec((1, _KT, l, h), lambda i, j: (i, j, 0, 0))
    node_spec = pl.BlockSpec((1, l, h), lambda i, j: (i, 0, 0))

    hv_out, p2, a2 = pl.pallas_call(
        functools.partial(_kernel_b, kt=_KT, n_kt=n_kt, ll=l),
        grid=grid,
        in_specs=[
            node_spec, edge_spec, edge_spec, node_spec,
            w_spec(w1b_t), w_spec(t("W2_w")), _row_spec(h),
            w_spec(t("W3_w")), _row_spec(h),
            w_spec(t("Win_w")), _row_spec(4 * h),
            w_spec(t("Wout_w")), _row_spec(h),
            _row_spec(h), _row_spec(h), _row_spec(h), _row_spec(h),
            w_spec(w11a_t), _row_spec(h),
            w_spec(w11c_t),
        ],
        out_specs=(node_spec, node_spec, node_spec),
        out_shape=(
            jax.ShapeDtypeStruct((b, l, h), jnp.float32),
            jax.ShapeDtypeStruct((b, l, h), jnp.float32),
            jax.ShapeDtypeStruct((b, l, h), jnp.float32),
        ),
        scratch_shapes=[pltpu.VMEM((l, h), jnp.float32)],
        compiler_params=pltpu.CompilerParams(
            dimension_semantics=("arbitrary", "arbitrary"),
        ),
    )(
        h_V, he_t, g1r, a1r,
        w1b_t, t("W2_w"), row("W2_b"), t("W3_w"), row("W3_b"),
        t("Win_w"), row("Win_b"), t("Wout_w"), row("Wout_b"),
        row("norm1_g"), row("norm1_b"), row("norm2_g"), row("norm2_b"),
        w11a_t, row("W11_b"), w11c_t,
    )

    g2 = _sc_gather(p2.reshape(n, h), idx_full, e_total, h, ch2, pad2)
    g2r = g2.reshape(b, k, l, h)

    he_out = pl.pallas_call(
        functools.partial(_kernel_c, kt=_KT, ll=l),
        grid=(b, n_kt),
        in_specs=[
            edge_spec, edge_spec, node_spec,
            w_spec(w11b_t), w_spec(t("W12_w")), _row_spec(h),
            w_spec(t("W13_w")), _row_spec(h),
            _row_spec(h), _row_spec(h),
        ],
        out_specs=edge_spec,
        out_shape=jax.ShapeDtypeStruct((b, k, l, h), jnp.float32),
        compiler_params=pltpu.CompilerParams(
            dimension_semantics=("parallel", "parallel"),
        ),
    )(
        he_t, g2r, a2,
        w11b_t, t("W12_w"), row("W12_b"), t("W13_w"), row("W13_b"),
        row("norm3_g"), row("norm3_b"),
    )

    return hv_out, he_out.transpose(0, 2, 1, 3)
